# Initial kernel scaffold; baseline (speedup 1.0000x reference)
import functools

import jax
import jax.numpy as jnp
from jax import lax
from jax.experimental import pallas as pl
from jax.experimental.pallas import tpu as pltpu
from jax.experimental.pallas import tpu_sc as plsc

_B = 4096
_F = 26
_D = 16
_ROWS_PER_FIELD = 40000
_NC = 2
_NS = 16
_NW = _NC * _NS
_BW = _B // _NW
_L = 16


def _body(x_hbm, table_hbm, w_hbm, b_hbm, out_hbm,
          xv, idxv, rowsv, wv, bv, accv, outv, sem):
    wid = lax.axis_index("s") * _NC + lax.axis_index("c")
    base = wid * _BW

    pltpu.sync_copy(x_hbm.at[pl.ds(base, _BW), :], xv)
    pltpu.sync_copy(w_hbm, wv)
    pltpu.sync_copy(b_hbm, bv)

    for f in range(_F):
        col = jnp.full((_L,), f, jnp.int32)
        for c in range(_BW // _L):
            rows = lax.iota(jnp.int32, _L) + (c * _L)
            vals = plsc.load_gather(xv, [rows, col]) + (f * _ROWS_PER_FIELD)
            idxv[f, pl.ds(c * _L, _L)] = vals

    copies = [
        pltpu.async_copy(table_hbm.at[idxv.at[f]], rowsv.at[f], sem)
        for f in range(_F)
    ]
    for c in copies:
        c.wait()

    def p1(i, carry):
        accs = [rowsv[f, i] * wv[f] for f in range(4)]
        for f in range(4, _F):
            accs[f % 4] = accs[f % 4] + rowsv[f, i] * wv[f]
        accv[i] = (accs[0] + accs[1]) + (accs[2] + accs[3])
        return carry

    lax.fori_loop(0, _BW, p1, 0)

    iot = lax.iota(jnp.int32, _L)
    for g in range(_BW // _L):
        rows16 = iot + (g * _L)
        tot = bv[...]
        for d in range(_D):
            col = jnp.full((_L,), d, jnp.int32)
            tot = tot + plsc.load_gather(accv, [rows16, col])
        outv[pl.ds(g * _L, _L)] = tot

    pltpu.sync_copy(outv, out_hbm.at[pl.ds(base, _BW)])


_sc_call = functools.partial(
    pl.kernel,
    out_type=jax.ShapeDtypeStruct((_B,), jnp.float32),
    mesh=plsc.VectorSubcoreMesh(core_axis_name="c", subcore_axis_name="s"),
    scratch_types=[
        pltpu.VMEM((_BW, _F), jnp.int32),
        pltpu.VMEM((_F, _BW), jnp.int32),
        pltpu.VMEM((_F, _BW, _D), jnp.float32),
        pltpu.VMEM((_F, _D), jnp.float32),
        pltpu.VMEM((_L,), jnp.float32),
        pltpu.VMEM((_BW, _D), jnp.float32),
        pltpu.VMEM((_BW,), jnp.float32),
        pltpu.SemaphoreType.DMA,
    ],
)(_body)


def kernel(x, table, W, b, current_epoch, current_step):
    w2 = W.reshape(_F, _D)
    b16 = jnp.broadcast_to(b.astype(jnp.float32), (_L,))
    out = _sc_call(x, table, w2, b16)
    return out.reshape(_B, 1)


# speedup vs baseline: 3.7130x; 3.7130x over previous
import functools

import jax
import jax.numpy as jnp
from jax import lax
from jax.experimental import pallas as pl
from jax.experimental.pallas import tpu as pltpu
from jax.experimental.pallas import tpu_sc as plsc

_B = 4096
_F = 26
_D = 16
_ROWS_PER_FIELD = 40000
_NC = 2
_NS = 16
_NW = _NC * _NS
_BW = _B // _NW
_L = 16


def _body(x_hbm, table_hbm, w_hbm, b_hbm, out_hbm,
          xv, idxv, rowsv, wv, bv, accv, outv, sem):
    wid = lax.axis_index("s") * _NC + lax.axis_index("c")
    base = wid * _BW

    pltpu.sync_copy(x_hbm.at[pl.ds(base * _F, _BW * _F)], xv)
    pltpu.sync_copy(w_hbm, wv)
    pltpu.sync_copy(b_hbm, bv)

    iot_f = lax.iota(jnp.int32, _L) * _F
    for f in range(_F):
        for c in range(_BW // _L):
            flat = iot_f + (c * _L * _F + f)
            vals = plsc.load_gather(xv, [flat]) + (f * _ROWS_PER_FIELD)
            idxv[f, pl.ds(c * _L, _L)] = vals

    copies = [
        pltpu.async_copy(table_hbm.at[idxv.at[f]], rowsv.at[f], sem)
        for f in range(_F)
    ]
    for c in copies:
        c.wait()

    def p1(i, carry):
        accs = [rowsv[f, i] * wv[f] for f in range(4)]
        for f in range(4, _F):
            accs[f % 4] = accs[f % 4] + rowsv[f, i] * wv[f]
        accv[pl.ds(i * _D, _D)] = (accs[0] + accs[1]) + (accs[2] + accs[3])
        return carry

    lax.fori_loop(0, _BW, p1, 0)

    iot_d = lax.iota(jnp.int32, _L) * _D
    for g in range(_BW // _L):
        tot = bv[...]
        for d in range(_D):
            tot = tot + plsc.load_gather(accv, [iot_d + (g * _L * _D + d)])
        outv[pl.ds(g * _L, _L)] = tot

    pltpu.sync_copy(outv, out_hbm.at[pl.ds(base, _BW)])


_sc_call = functools.partial(
    pl.kernel,
    out_type=jax.ShapeDtypeStruct((_B,), jnp.float32),
    mesh=plsc.VectorSubcoreMesh(core_axis_name="c", subcore_axis_name="s"),
    compiler_params=pltpu.CompilerParams(
        needs_layout_passes=False, use_tc_tiling_on_sc=False),
    scratch_types=[
        pltpu.VMEM((_BW * _F,), jnp.int32),
        pltpu.VMEM((_F, _BW), jnp.int32),
        pltpu.VMEM((_F, _BW, _D), jnp.float32),
        pltpu.VMEM((_F, _D), jnp.float32),
        pltpu.VMEM((_L,), jnp.float32),
        pltpu.VMEM((_BW * _D,), jnp.float32),
        pltpu.VMEM((_BW,), jnp.float32),
        pltpu.SemaphoreType.DMA,
    ],
)(_body)


def kernel(x, table, W, b, current_epoch, current_step):
    w2 = W.reshape(_F, _D)
    b16 = jnp.broadcast_to(b.astype(jnp.float32), (_L,))
    out = _sc_call(x.reshape(_B * _F), table, w2, b16)
    return out.reshape(_B, 1)
